# baseline (speedup 1.0000x reference)
import functools
import math

import jax
import jax.numpy as jnp
from jax import lax
from jax.experimental import pallas as pl
from jax.experimental.pallas import tpu as pltpu
from jax.experimental.pallas import tpu_sc as plsc

_B, _T, _D, _H, _CS, _DV = 16, 4096, 128, 128, 64, 64
_TILE = 256
_NT = _T // _TILE
_RPB = 2
_ZP = 128
_TP = _T + _ZP
_GC = 128


def _pos_enc(L, Hd):
    pos = jnp.arange(L, dtype=jnp.float32)[:, None]
    div = jnp.exp(jnp.arange(0, Hd, 2, dtype=jnp.float32) * (-math.log(10000.0) / Hd))
    pe = jnp.zeros((L, Hd), dtype=jnp.float32)
    pe = pe.at[:, 0::2].set(jnp.sin(pos * div))
    pe = pe.at[:, 1::2].set(jnp.cos(pos * div))
    return pe


@functools.lru_cache(maxsize=1)
def _consts():
    n = _T // _CS
    pe = _pos_enc(_T, _H)
    gs = []
    for i in range(2):
        g = jax.random.gumbel(jax.random.key(100 + i), (_B * n, _CS, 2), jnp.float32)
        gs.append(g.reshape(_B, _T, 2))
    g4 = jnp.concatenate(gs, axis=-1)
    return jax.device_put(pe), jax.device_put(g4)


def _tc_body(data_ref, v0_ref, v1_ref, we_ref, be_ref, w4_ref, bf4_ref,
             pe_ref, g4_ref, h_ref, z_ref):
    bf4 = bf4_ref[...]
    vv = [(jnp.broadcast_to(v0_ref[r, 0:1, :], (_TILE, _DV)),
           jnp.broadcast_to(v1_ref[r, 0:1, :], (_TILE, _DV)))
          for r in range(_RPB)]

    for r in range(_RPB):
        h_ref[r * _TP + _T:(r + 1) * _TP, :] = jnp.zeros((_ZP, _H), jnp.float32)

    for k in range(_NT):
        psl = slice(k * _TILE, (k + 1) * _TILE)
        for r in range(_RPB):
            sl = slice(r * _T + k * _TILE, r * _T + (k + 1) * _TILE)
            dk = data_ref[sl, :]
            hk = ((jnp.dot(dk, we_ref[...], preferred_element_type=jnp.float32)
                   + be_ref[...][None, :]) + pe_ref[psl, :]) * 8.0
            feat = jnp.concatenate([hk, vv[r][0], vv[r][1]], axis=1)
            lg = jnp.dot(feat, w4_ref[...], preferred_element_type=jnp.float32)
            z = g4_ref[r, psl, :] + (lg + bf4)
            h_ref[r * _TP + k * _TILE:r * _TP + (k + 1) * _TILE, :] = hk
            z_ref[sl, :] = z


def _tc_call(data, v0, v1, W_embed, b_embed, w4, bf4, pe, g4):
    grid = (_B // _RPB,)
    return pl.pallas_call(
        _tc_body,
        grid=grid,
        in_specs=[
            pl.BlockSpec((_RPB * _T, _D), lambda b: (b, 0)),
            pl.BlockSpec((_RPB, 1, _DV), lambda b: (b, 0, 0)),
            pl.BlockSpec((_RPB, 1, _DV), lambda b: (b, 0, 0)),
            pl.BlockSpec((_D, _H), lambda b: (0, 0)),
            pl.BlockSpec((_H,), lambda b: (0,)),
            pl.BlockSpec((_H + 2 * _DV, 4), lambda b: (0, 0)),
            pl.BlockSpec((1, 4), lambda b: (0, 0)),
            pl.BlockSpec((_T, _H), lambda b: (0, 0)),
            pl.BlockSpec((_RPB, _T, 4), lambda b: (b, 0, 0)),
        ],
        out_specs=[
            pl.BlockSpec((_RPB * _TP, _H), lambda b: (b, 0)),
            pl.BlockSpec((_RPB * _T, 4), lambda b: (b, 0)),
        ],
        out_shape=[
            jax.ShapeDtypeStruct((_B * _TP, _H), jnp.float32),
            jax.ShapeDtypeStruct((_B * _T, 4), jnp.float32),
        ],
    )(data.reshape(_B * _T, _D), v0, v1, W_embed, b_embed, w4, bf4, pe, g4)


def _sc_kernel(hpad, z2):
    mesh = plsc.VectorSubcoreMesh(core_axis_name="c", subcore_axis_name="s")
    info = plsc.get_sparse_core_info()
    nc = info.num_cores

    @functools.partial(
        pl.kernel,
        mesh=mesh,
        compiler_params=pltpu.CompilerParams(needs_layout_passes=False),
        out_type=jax.ShapeDtypeStruct((_B * _T, _H), jnp.float32),
        scratch_types=[
            pltpu.VMEM((4 * _TILE,), jnp.float32),
            pltpu.VMEM((_T,), jnp.int32),
            pltpu.VMEM((_GC, _H), jnp.float32),
            pltpu.SemaphoreType.DMA,
            pltpu.SemaphoreType.DMA,
        ],
    )
    def k(hpad_hbm, z_hbm, out_hbm, zbuf, idxbuf, rows, zsem, gsem):
        wid = lax.axis_index("s") * nc + lax.axis_index("c")

        @pl.when(wid < _B)
        def _():
            row0 = wid * _T
            hbase = wid * _TP
            zrow = hbase + _T
            lanes = lax.iota(jnp.int32, 16)

            def init_body(i, _):
                idxbuf[pl.ds(i * 16, 16)] = jnp.full((16,), zrow, jnp.int32)
                return 0
            lax.fori_loop(0, _T // 16, init_body, 0)

            def blk_body(blk, cnt):
                pltpu.sync_copy(
                    z_hbm.at[pl.ds(row0 * 4 + blk * 4 * _TILE, 4 * _TILE)], zbuf)

                def grp_body(g, cnt):
                    base = g * 64
                    z0 = plsc.load_gather(zbuf, [lanes * 4 + base])
                    z1 = plsc.load_gather(zbuf, [lanes * 4 + (base + 1)])
                    z2v = plsc.load_gather(zbuf, [lanes * 4 + (base + 2)])
                    z3 = plsc.load_gather(zbuf, [lanes * 4 + (base + 3)])
                    m = jnp.logical_and(z0 >= z1, z2v >= z3)
                    mi = m.astype(jnp.int32)
                    pos = cnt + plsc.cumsum(mi) - 1
                    tok = hbase + blk * _TILE + g * 16 + lanes
                    plsc.store_scatter(idxbuf, [pos], tok, mask=m)
                    return cnt + plsc.all_reduce_population_count(m)

                return lax.fori_loop(0, _TILE // 16, grp_body, cnt)

            lax.fori_loop(0, _NT, blk_body, jnp.zeros((16,), jnp.int32))

            def gat_body(j, _):
                pltpu.async_copy(
                    hpad_hbm.at[idxbuf.at[pl.ds(j * _GC, _GC)]], rows, gsem).wait()
                pltpu.sync_copy(rows, out_hbm.at[pl.ds(row0 + j * _GC, _GC)])
                return 0
            lax.fori_loop(0, _T // _GC, gat_body, 0)

    return k(hpad, z2)


def kernel(data, value_0, value_1, W_embed, b_embed, W_f, b_f):
    pe, g4 = _consts()
    v0 = value_0.reshape(_B, 1, _DV)
    v1 = value_1.reshape(_B, 1, _DV)
    wh = W_f[:_H, :]
    wv = W_f[_H:, :]
    zv = jnp.zeros_like(wv)
    w4 = jnp.concatenate(
        [jnp.concatenate([wh, wv, zv], axis=0),
         jnp.concatenate([wh, zv, wv], axis=0)], axis=1)
    bf4 = jnp.concatenate([b_f, b_f]).reshape(1, 4)

    hpad, z = _tc_call(data, v0, v1, W_embed, b_embed, w4, bf4, pe, g4)
    out = _sc_kernel(hpad, z.reshape(_B * _T * 4))
    return out.reshape(_B, _T, _H)


# speedup vs baseline: 1.0822x; 1.0822x over previous
import functools
import math

import jax
import jax.numpy as jnp
from jax import lax
from jax.experimental import pallas as pl
from jax.experimental.pallas import tpu as pltpu
from jax.experimental.pallas import tpu_sc as plsc

_B, _T, _D, _H, _CS, _DV = 16, 4096, 128, 128, 64, 64
_TILE = 256
_NT = _T // _TILE
_RPB = 2
_ZP = 128
_TP = _T + _ZP
_GC = 128
_NB = 4


def _pos_enc(L, Hd):
    pos = jnp.arange(L, dtype=jnp.float32)[:, None]
    div = jnp.exp(jnp.arange(0, Hd, 2, dtype=jnp.float32) * (-math.log(10000.0) / Hd))
    pe = jnp.zeros((L, Hd), dtype=jnp.float32)
    pe = pe.at[:, 0::2].set(jnp.sin(pos * div))
    pe = pe.at[:, 1::2].set(jnp.cos(pos * div))
    return pe


@functools.lru_cache(maxsize=1)
def _consts():
    n = _T // _CS
    pe = _pos_enc(_T, _H)
    gs = []
    for i in range(2):
        g = jax.random.gumbel(jax.random.key(100 + i), (_B * n, _CS, 2), jnp.float32)
        gs.append(g.reshape(_B, _T, 2))
    g4 = jnp.concatenate(gs, axis=-1)
    return jax.device_put(pe), jax.device_put(g4)


def _tc_body(data_ref, v0_ref, v1_ref, we_ref, be_ref, w4_ref, bf4_ref,
             pe_ref, g4_ref, h_ref, z_ref):
    bf4 = bf4_ref[...]
    vv = [(jnp.broadcast_to(v0_ref[r, 0:1, :], (_TILE, _DV)),
           jnp.broadcast_to(v1_ref[r, 0:1, :], (_TILE, _DV)))
          for r in range(_RPB)]

    for r in range(_RPB):
        h_ref[r * _TP + _T:(r + 1) * _TP, :] = jnp.zeros((_ZP, _H), jnp.float32)

    for k in range(_NT):
        psl = slice(k * _TILE, (k + 1) * _TILE)
        for r in range(_RPB):
            sl = slice(r * _T + k * _TILE, r * _T + (k + 1) * _TILE)
            dk = data_ref[sl, :]
            hk = ((jnp.dot(dk, we_ref[...], preferred_element_type=jnp.float32)
                   + be_ref[...][None, :]) + pe_ref[psl, :]) * 8.0
            feat = jnp.concatenate([hk, vv[r][0], vv[r][1]], axis=1)
            lg = jnp.dot(feat, w4_ref[...], preferred_element_type=jnp.float32)
            z = g4_ref[r, psl, :] + (lg + bf4)
            h_ref[r * _TP + k * _TILE:r * _TP + (k + 1) * _TILE, :] = hk
            z_ref[sl, :] = z


def _tc_call(data, v0, v1, W_embed, b_embed, w4, bf4, pe, g4):
    grid = (_B // _RPB,)
    return pl.pallas_call(
        _tc_body,
        grid=grid,
        in_specs=[
            pl.BlockSpec((_RPB * _T, _D), lambda b: (b, 0)),
            pl.BlockSpec((_RPB, 1, _DV), lambda b: (b, 0, 0)),
            pl.BlockSpec((_RPB, 1, _DV), lambda b: (b, 0, 0)),
            pl.BlockSpec((_D, _H), lambda b: (0, 0)),
            pl.BlockSpec((_H,), lambda b: (0,)),
            pl.BlockSpec((_H + 2 * _DV, 4), lambda b: (0, 0)),
            pl.BlockSpec((1, 4), lambda b: (0, 0)),
            pl.BlockSpec((_T, _H), lambda b: (0, 0)),
            pl.BlockSpec((_RPB, _T, 4), lambda b: (b, 0, 0)),
        ],
        out_specs=[
            pl.BlockSpec((_RPB * _TP, _H), lambda b: (b, 0)),
            pl.BlockSpec((_RPB * _T, 4), lambda b: (b, 0)),
        ],
        out_shape=[
            jax.ShapeDtypeStruct((_B * _TP, _H), jnp.float32),
            jax.ShapeDtypeStruct((_B * _T, 4), jnp.float32),
        ],
    )(data.reshape(_B * _T, _D), v0, v1, W_embed, b_embed, w4, bf4, pe, g4)


def _sc_kernel(hpad, z2):
    mesh = plsc.VectorSubcoreMesh(core_axis_name="c", subcore_axis_name="s")
    info = plsc.get_sparse_core_info()
    nc = info.num_cores

    @functools.partial(
        pl.kernel,
        mesh=mesh,
        compiler_params=pltpu.CompilerParams(needs_layout_passes=False),
        out_type=jax.ShapeDtypeStruct((_B * _T, _H), jnp.float32),
        scratch_types=[
            pltpu.VMEM((4 * _TILE,), jnp.float32),
            pltpu.VMEM((_T,), jnp.int32),
            pltpu.VMEM((_NB, _GC, _H), jnp.float32),
            [pltpu.SemaphoreType.DMA] * _NB,
            [pltpu.SemaphoreType.DMA] * _NB,
        ],
    )
    def k(hpad_hbm, z_hbm, out_hbm, zbuf, idxbuf, rows, gsems, osems):
        wid = lax.axis_index("s") * nc + lax.axis_index("c")

        @pl.when(wid < _B)
        def _():
            row0 = wid * _T
            hbase = wid * _TP
            zrow = hbase + _T
            lanes = lax.iota(jnp.int32, 16)

            def init_body(i, _):
                idxbuf[pl.ds(i * 16, 16)] = jnp.full((16,), zrow, jnp.int32)
                return 0
            lax.fori_loop(0, _T // 16, init_body, 0)

            def blk_body(blk, cnt):
                pltpu.sync_copy(
                    z_hbm.at[pl.ds(row0 * 4 + blk * 4 * _TILE, 4 * _TILE)], zbuf)

                def grp_body(g, cnt):
                    base = g * 64
                    z0 = plsc.load_gather(zbuf, [lanes * 4 + base])
                    z1 = plsc.load_gather(zbuf, [lanes * 4 + (base + 1)])
                    z2v = plsc.load_gather(zbuf, [lanes * 4 + (base + 2)])
                    z3 = plsc.load_gather(zbuf, [lanes * 4 + (base + 3)])
                    m = jnp.logical_and(z0 >= z1, z2v >= z3)
                    mi = m.astype(jnp.int32)
                    pos = cnt + plsc.cumsum(mi) - 1
                    tok = hbase + blk * _TILE + g * 16 + lanes
                    plsc.store_scatter(idxbuf, [pos], tok, mask=m)
                    return cnt + plsc.all_reduce_population_count(m)

                return lax.fori_loop(0, _TILE // 16, grp_body, cnt)

            lax.fori_loop(0, _NT, blk_body, jnp.zeros((16,), jnp.int32))

            nj = _T // _GC
            gcopies = [None] * nj
            ocopies = [None] * nj

            def gat(j):
                return pltpu.make_async_copy(
                    hpad_hbm.at[idxbuf.at[pl.ds(j * _GC, _GC)]],
                    rows.at[j % _NB], gsems[j % _NB])

            for j in range(_NB):
                gcopies[j] = gat(j)
                gcopies[j].start()
            for j in range(nj):
                s = j % _NB
                gcopies[j].wait()
                ocopies[j] = pltpu.make_async_copy(
                    rows.at[s], out_hbm.at[pl.ds(row0 + j * _GC, _GC)], osems[s])
                ocopies[j].start()
                if j + _NB < nj:
                    ocopies[j].wait()
                    gcopies[j + _NB] = gat(j + _NB)
                    gcopies[j + _NB].start()
            for j in range(nj - _NB, nj):
                ocopies[j].wait()

    return k(hpad, z2)


def kernel(data, value_0, value_1, W_embed, b_embed, W_f, b_f):
    pe, g4 = _consts()
    v0 = value_0.reshape(_B, 1, _DV)
    v1 = value_1.reshape(_B, 1, _DV)
    wh = W_f[:_H, :]
    wv = W_f[_H:, :]
    zv = jnp.zeros_like(wv)
    w4 = jnp.concatenate(
        [jnp.concatenate([wh, wv, zv], axis=0),
         jnp.concatenate([wh, zv, wv], axis=0)], axis=1)
    bf4 = jnp.concatenate([b_f, b_f]).reshape(1, 4)

    hpad, z = _tc_call(data, v0, v1, W_embed, b_embed, w4, bf4, pe, g4)
    out = _sc_kernel(hpad, z.reshape(_B * _T * 4))
    return out.reshape(_B, _T, _H)


# speedup vs baseline: 1.0825x; 1.0003x over previous
import functools
import math

import jax
import jax.numpy as jnp
from jax import lax
from jax.experimental import pallas as pl
from jax.experimental.pallas import tpu as pltpu
from jax.experimental.pallas import tpu_sc as plsc

_B, _T, _D, _H, _CS, _DV = 16, 4096, 128, 128, 64, 64
_TILE = 256
_NT = _T // _TILE
_RPB = 2
_ZP = 128
_TP = _T + _ZP
_GC = 128
_NB = 4


def _pos_enc(L, Hd):
    pos = jnp.arange(L, dtype=jnp.float32)[:, None]
    div = jnp.exp(jnp.arange(0, Hd, 2, dtype=jnp.float32) * (-math.log(10000.0) / Hd))
    pe = jnp.zeros((L, Hd), dtype=jnp.float32)
    pe = pe.at[:, 0::2].set(jnp.sin(pos * div))
    pe = pe.at[:, 1::2].set(jnp.cos(pos * div))
    return pe


@functools.lru_cache(maxsize=1)
def _consts():
    n = _T // _CS
    pe = _pos_enc(_T, _H)
    gs = []
    for i in range(2):
        g = jax.random.gumbel(jax.random.key(100 + i), (_B * n, _CS, 2), jnp.float32)
        gs.append(g.reshape(_B, _T, 2))
    g4 = jnp.concatenate(gs, axis=-1)
    return jax.device_put(pe), jax.device_put(g4)


def _tc_body(data_ref, v0_ref, v1_ref, we_ref, be_ref, w4_ref, bf4_ref,
             pe_ref, g4_ref, h_ref, z_ref):
    bf4 = bf4_ref[...]
    vv = [(jnp.broadcast_to(v0_ref[r, 0:1, :], (_TILE, _DV)),
           jnp.broadcast_to(v1_ref[r, 0:1, :], (_TILE, _DV)))
          for r in range(_RPB)]

    for r in range(_RPB):
        h_ref[r * _TP + _T:(r + 1) * _TP, :] = jnp.zeros((_ZP, _H), jnp.float32)

    for k in range(_NT):
        psl = slice(k * _TILE, (k + 1) * _TILE)
        for r in range(_RPB):
            sl = slice(r * _T + k * _TILE, r * _T + (k + 1) * _TILE)
            dk = data_ref[sl, :]
            hk = ((jnp.dot(dk, we_ref[...], preferred_element_type=jnp.float32)
                   + be_ref[...][None, :]) + pe_ref[psl, :]) * 8.0
            feat = jnp.concatenate([hk, vv[r][0], vv[r][1]], axis=1)
            lg = jnp.dot(feat, w4_ref[...], preferred_element_type=jnp.float32)
            z = g4_ref[r, psl, :] + (lg + bf4)
            h_ref[r * _TP + k * _TILE:r * _TP + (k + 1) * _TILE, :] = hk
            z_ref[sl, :] = z


def _tc_call(data, v0, v1, W_embed, b_embed, w4, bf4, pe, g4):
    grid = (_B // _RPB,)
    return pl.pallas_call(
        _tc_body,
        grid=grid,
        in_specs=[
            pl.BlockSpec((_RPB * _T, _D), lambda b: (b, 0)),
            pl.BlockSpec((_RPB, 1, _DV), lambda b: (b, 0, 0)),
            pl.BlockSpec((_RPB, 1, _DV), lambda b: (b, 0, 0)),
            pl.BlockSpec((_D, _H), lambda b: (0, 0)),
            pl.BlockSpec((_H,), lambda b: (0,)),
            pl.BlockSpec((_H + 2 * _DV, 4), lambda b: (0, 0)),
            pl.BlockSpec((1, 4), lambda b: (0, 0)),
            pl.BlockSpec((_T, _H), lambda b: (0, 0)),
            pl.BlockSpec((_RPB, _T, 4), lambda b: (b, 0, 0)),
        ],
        out_specs=[
            pl.BlockSpec((_RPB * _TP, _H), lambda b: (b, 0)),
            pl.BlockSpec((_RPB * _T, 4), lambda b: (b, 0)),
        ],
        out_shape=[
            jax.ShapeDtypeStruct((_B * _TP, _H), jnp.float32),
            jax.ShapeDtypeStruct((_B * _T, 4), jnp.float32),
        ],
    )(data.reshape(_B * _T, _D), v0, v1, W_embed, b_embed, w4, bf4, pe, g4)


def _sc_kernel(hpad, z2):
    mesh = plsc.VectorSubcoreMesh(core_axis_name="c", subcore_axis_name="s")
    info = plsc.get_sparse_core_info()
    nc = info.num_cores

    @functools.partial(
        pl.kernel,
        mesh=mesh,
        compiler_params=pltpu.CompilerParams(needs_layout_passes=False),
        out_type=jax.ShapeDtypeStruct((_B * _T, _H), jnp.float32),
        scratch_types=[
            pltpu.VMEM((4 * _TILE,), jnp.float32),
            pltpu.VMEM((_T,), jnp.int32),
            pltpu.VMEM((_NB, _GC, _H), jnp.float32),
            [pltpu.SemaphoreType.DMA] * _NB,
            [pltpu.SemaphoreType.DMA] * _NB,
        ],
    )
    def k(hpad_hbm, z_hbm, out_hbm, zbuf, idxbuf, rows, gsems, osems):
        wid = lax.axis_index("s") * nc + lax.axis_index("c")

        @pl.when(wid < _B)
        def _():
            row0 = wid * _T
            hbase = wid * _TP
            zrow = hbase + _T
            lanes = lax.iota(jnp.int32, 16)

            zfill = jnp.full((16,), zrow, jnp.int32)

            def init_body(i, _):
                for u in range(8):
                    idxbuf[pl.ds(i * 128 + u * 16, 16)] = zfill
                return 0
            lax.fori_loop(0, _T // 128, init_body, 0)

            lanes4 = lanes * 4

            def blk_body(blk, cnt):
                pltpu.sync_copy(
                    z_hbm.at[pl.ds(row0 * 4 + blk * 4 * _TILE, 4 * _TILE)], zbuf)
                for g in range(_TILE // 16):
                    base = g * 64
                    z0 = plsc.load_gather(zbuf, [lanes4 + base])
                    z1 = plsc.load_gather(zbuf, [lanes4 + (base + 1)])
                    z2v = plsc.load_gather(zbuf, [lanes4 + (base + 2)])
                    z3 = plsc.load_gather(zbuf, [lanes4 + (base + 3)])
                    m = jnp.logical_and(z0 >= z1, z2v >= z3)
                    mi = m.astype(jnp.int32)
                    pos = cnt + plsc.cumsum(mi) - 1
                    tok = hbase + blk * _TILE + g * 16 + lanes
                    plsc.store_scatter(idxbuf, [pos], tok, mask=m)
                    cnt = cnt + plsc.all_reduce_population_count(m)
                return cnt

            lax.fori_loop(0, _NT, blk_body, jnp.zeros((16,), jnp.int32))

            nj = _T // _GC
            gcopies = [None] * nj
            ocopies = [None] * nj

            def gat(j):
                return pltpu.make_async_copy(
                    hpad_hbm.at[idxbuf.at[pl.ds(j * _GC, _GC)]],
                    rows.at[j % _NB], gsems[j % _NB])

            for j in range(_NB):
                gcopies[j] = gat(j)
                gcopies[j].start()
            for j in range(nj):
                s = j % _NB
                gcopies[j].wait()
                ocopies[j] = pltpu.make_async_copy(
                    rows.at[s], out_hbm.at[pl.ds(row0 + j * _GC, _GC)], osems[s])
                ocopies[j].start()
                if j + _NB < nj:
                    ocopies[j].wait()
                    gcopies[j + _NB] = gat(j + _NB)
                    gcopies[j + _NB].start()
            for j in range(nj - _NB, nj):
                ocopies[j].wait()

    return k(hpad, z2)


def kernel(data, value_0, value_1, W_embed, b_embed, W_f, b_f):
    pe, g4 = _consts()
    v0 = value_0.reshape(_B, 1, _DV)
    v1 = value_1.reshape(_B, 1, _DV)
    wh = W_f[:_H, :]
    wv = W_f[_H:, :]
    zv = jnp.zeros_like(wv)
    w4 = jnp.concatenate(
        [jnp.concatenate([wh, wv, zv], axis=0),
         jnp.concatenate([wh, zv, wv], axis=0)], axis=1)
    bf4 = jnp.concatenate([b_f, b_f]).reshape(1, 4)

    hpad, z = _tc_call(data, v0, v1, W_embed, b_embed, w4, bf4, pe, g4)
    out = _sc_kernel(hpad, z.reshape(_B * _T * 4))
    return out.reshape(_B, _T, _H)


# speedup vs baseline: 2.1704x; 2.0049x over previous
import functools
import math

import jax
import jax.numpy as jnp
from jax import lax
from jax.experimental import pallas as pl
from jax.experimental.pallas import tpu as pltpu
from jax.experimental.pallas import tpu_sc as plsc

_B, _T, _D, _H, _CS, _DV = 16, 4096, 128, 128, 64, 64
_TILE = 256
_NT = _T // _TILE
_RPB = 2
_ZP = 128
_TP = _T + _ZP
_GC = 128
_NB = 4


def _pos_enc(L, Hd):
    pos = jnp.arange(L, dtype=jnp.float32)[:, None]
    div = jnp.exp(jnp.arange(0, Hd, 2, dtype=jnp.float32) * (-math.log(10000.0) / Hd))
    pe = jnp.zeros((L, Hd), dtype=jnp.float32)
    pe = pe.at[:, 0::2].set(jnp.sin(pos * div))
    pe = pe.at[:, 1::2].set(jnp.cos(pos * div))
    return pe


@functools.lru_cache(maxsize=1)
def _consts():
    n = _T // _CS
    pe = _pos_enc(_T, _H)
    gs = []
    for i in range(2):
        g = jax.random.gumbel(jax.random.key(100 + i), (_B * n, _CS, 2), jnp.float32)
        gs.append(g.reshape(_B, _T, 2))
    g4 = jnp.concatenate(gs, axis=-1)
    return jax.device_put(pe), jax.device_put(g4)


def _tc_body(data_ref, v0_ref, v1_ref, we_ref, be_ref, w4_ref, bf4_ref,
             pe_ref, g4_ref, h_ref, z_ref):
    bf4 = bf4_ref[...]
    vv = [(jnp.broadcast_to(v0_ref[r, 0:1, :], (_TILE, _DV)),
           jnp.broadcast_to(v1_ref[r, 0:1, :], (_TILE, _DV)))
          for r in range(_RPB)]

    for r in range(_RPB):
        h_ref[r * _TP + _T:(r + 1) * _TP, :] = jnp.zeros((_ZP, _H), jnp.float32)

    for k in range(_NT):
        psl = slice(k * _TILE, (k + 1) * _TILE)
        for r in range(_RPB):
            sl = slice(r * _T + k * _TILE, r * _T + (k + 1) * _TILE)
            dk = data_ref[sl, :]
            hk = ((jnp.dot(dk, we_ref[...], preferred_element_type=jnp.float32)
                   + be_ref[...][None, :]) + pe_ref[psl, :]) * 8.0
            feat = jnp.concatenate([hk, vv[r][0], vv[r][1]], axis=1)
            lg = jnp.dot(feat, w4_ref[...], preferred_element_type=jnp.float32)
            z = g4_ref[r, psl, :] + (lg + bf4)
            h_ref[r * _TP + k * _TILE:r * _TP + (k + 1) * _TILE, :] = hk
            z_ref[sl, :] = z


def _tc_call(data, v0, v1, W_embed, b_embed, w4, bf4, pe, g4):
    grid = (_B // _RPB,)
    return pl.pallas_call(
        _tc_body,
        grid=grid,
        in_specs=[
            pl.BlockSpec((_RPB * _T, _D), lambda b: (b, 0)),
            pl.BlockSpec((_RPB, 1, _DV), lambda b: (b, 0, 0)),
            pl.BlockSpec((_RPB, 1, _DV), lambda b: (b, 0, 0)),
            pl.BlockSpec((_D, _H), lambda b: (0, 0)),
            pl.BlockSpec((_H,), lambda b: (0,)),
            pl.BlockSpec((_H + 2 * _DV, 4), lambda b: (0, 0)),
            pl.BlockSpec((1, 4), lambda b: (0, 0)),
            pl.BlockSpec((_T, _H), lambda b: (0, 0)),
            pl.BlockSpec((_RPB, _T, 4), lambda b: (b, 0, 0)),
        ],
        out_specs=[
            pl.BlockSpec((_RPB * _TP, _H), lambda b: (b, 0)),
            pl.BlockSpec((_RPB * _T, 4), lambda b: (b, 0)),
        ],
        out_shape=[
            jax.ShapeDtypeStruct((_B * _TP, _H), jnp.float32),
            jax.ShapeDtypeStruct((_B * _T, 4), jnp.float32),
        ],
    )(data.reshape(_B * _T, _D), v0, v1, W_embed, b_embed, w4, bf4, pe, g4)


def _sc_kernel(hpad, z2):
    mesh = plsc.VectorSubcoreMesh(core_axis_name="c", subcore_axis_name="s")
    info = plsc.get_sparse_core_info()
    nc = info.num_cores

    @functools.partial(
        pl.kernel,
        mesh=mesh,
        compiler_params=pltpu.CompilerParams(needs_layout_passes=False),
        out_type=jax.ShapeDtypeStruct((_B * _T, _H), jnp.float32),
        scratch_types=[
            pltpu.VMEM((4 * _TILE,), jnp.float32),
            pltpu.VMEM((_T,), jnp.int32),
            pltpu.VMEM((_NB, _GC, _H), jnp.float32),
            pltpu.VMEM((_GC, _H), jnp.float32),
            [pltpu.SemaphoreType.DMA] * _NB,
            [pltpu.SemaphoreType.DMA] * _NB,
            pltpu.SemaphoreType.DMA,
        ],
    )
    def k(hpad_hbm, z_hbm, out_hbm, zbuf, idxbuf, rows, zerobuf, gsems, osems,
          zosem):
        wid = lax.axis_index("s") * nc + lax.axis_index("c")

        @pl.when(wid < _B)
        def _():
            row0 = wid * _T
            hbase = wid * _TP
            zrow = hbase + _T
            lanes = lax.iota(jnp.int32, 16)

            zfill = jnp.full((16,), zrow, jnp.int32)

            def init_body(i, _):
                for u in range(8):
                    idxbuf[pl.ds(i * 128 + u * 16, 16)] = zfill
                return 0
            lax.fori_loop(0, _T // 128, init_body, 0)

            lanes4 = lanes * 4

            def blk_body(blk, cnt):
                pltpu.sync_copy(
                    z_hbm.at[pl.ds(row0 * 4 + blk * 4 * _TILE, 4 * _TILE)], zbuf)
                for g in range(_TILE // 16):
                    base = g * 64
                    z0 = plsc.load_gather(zbuf, [lanes4 + base])
                    z1 = plsc.load_gather(zbuf, [lanes4 + (base + 1)])
                    z2v = plsc.load_gather(zbuf, [lanes4 + (base + 2)])
                    z3 = plsc.load_gather(zbuf, [lanes4 + (base + 3)])
                    m = jnp.logical_and(z0 >= z1, z2v >= z3)
                    mi = m.astype(jnp.int32)
                    pos = cnt + plsc.cumsum(mi) - 1
                    tok = hbase + blk * _TILE + g * 16 + lanes
                    plsc.store_scatter(idxbuf, [pos], tok, mask=m)
                    cnt = cnt + plsc.all_reduce_population_count(m)
                return cnt

            cntf = lax.fori_loop(0, _NT, blk_body, jnp.zeros((16,), jnp.int32))
            ks = jnp.max(cntf)

            nj = _T // _GC

            def vpred(j):
                return j * _GC < ks

            def gstart(j):
                pltpu.make_async_copy(
                    hpad_hbm.at[idxbuf.at[pl.ds(j * _GC, _GC)]],
                    rows.at[j % _NB], gsems[j % _NB]).start()

            def gwait(j):
                pltpu.make_async_copy(
                    hpad_hbm.at[idxbuf.at[pl.ds(j * _GC, _GC)]],
                    rows.at[j % _NB], gsems[j % _NB]).wait()

            def ocopy(j):
                return pltpu.make_async_copy(
                    rows.at[j % _NB],
                    out_hbm.at[pl.ds(row0 + j * _GC, _GC)], osems[j % _NB])

            def zcopy(j):
                return pltpu.make_async_copy(
                    zerobuf, out_hbm.at[pl.ds(row0 + j * _GC, _GC)], zosem)

            pltpu.sync_copy(hpad_hbm.at[pl.ds(zrow, _GC)], zerobuf)

            for j in range(_NB):
                pl.when(vpred(j))(functools.partial(gstart, j))
            for j in range(nj):
                @pl.when(vpred(j))
                def _(j=j):
                    gwait(j)
                    ocopy(j).start()
                if j + _NB < nj:
                    @pl.when(vpred(j + _NB))
                    def _(j=j):
                        ocopy(j).wait()
                        gstart(j + _NB)
            for j in range(nj):
                done_in_loop = vpred(j + _NB) if j + _NB < nj else False
                @pl.when(jnp.logical_and(vpred(j),
                                         jnp.logical_not(done_in_loop)))
                def _(j=j):
                    ocopy(j).wait()
            for j in range(nj):
                pl.when(jnp.logical_not(vpred(j)))(
                    functools.partial(lambda j: zcopy(j).start(), j))
            for j in range(nj):
                pl.when(jnp.logical_not(vpred(j)))(
                    functools.partial(lambda j: zcopy(j).wait(), j))

    return k(hpad, z2)


def kernel(data, value_0, value_1, W_embed, b_embed, W_f, b_f):
    pe, g4 = _consts()
    v0 = value_0.reshape(_B, 1, _DV)
    v1 = value_1.reshape(_B, 1, _DV)
    wh = W_f[:_H, :]
    wv = W_f[_H:, :]
    zv = jnp.zeros_like(wv)
    w4 = jnp.concatenate(
        [jnp.concatenate([wh, wv, zv], axis=0),
         jnp.concatenate([wh, zv, wv], axis=0)], axis=1)
    bf4 = jnp.concatenate([b_f, b_f]).reshape(1, 4)

    hpad, z = _tc_call(data, v0, v1, W_embed, b_embed, w4, bf4, pe, g4)
    out = _sc_kernel(hpad, z.reshape(_B * _T * 4))
    return out.reshape(_B, _T, _H)
